# Initial kernel scaffold; baseline (speedup 1.0000x reference)
import jax
import jax.numpy as jnp
from jax.experimental import pallas as pl


def kernel(x, W, b):
    raise NotImplementedError("write your pallas kernel here")


# speedup vs baseline: 7.0112x; 7.0112x over previous
import jax
import jax.numpy as jnp
from jax.experimental import pallas as pl

MODEL_DIM = 4096
NUM_EXPERTS = 64
TOP_K = 8
N_TOKENS = 32768

BLOCK_N = 1024


def _body(x_ref, w_ref, b_ref, o_ref):
    logits = jnp.dot(x_ref[...], w_ref[...], preferred_element_type=jnp.float32)
    logits = logits + b_ref[...]
    neg_inf = jnp.float32(-jnp.inf)
    v = logits
    rowmax = None
    thr = None
    for i in range(TOP_K):
        cur = jnp.max(v, axis=-1, keepdims=True)
        if i == 0:
            rowmax = cur
        thr = cur
        if i != TOP_K - 1:
            v = jnp.where(v >= cur, neg_inf, v)
    w = jnp.where(logits >= thr, jnp.exp(logits - rowmax), jnp.float32(0.0))
    o_ref[...] = w / jnp.sum(w, axis=-1, keepdims=True)


def kernel(x, W, b):
    b2 = b.reshape(1, NUM_EXPERTS)
    grid = (N_TOKENS // BLOCK_N,)
    return pl.pallas_call(
        _body,
        grid=grid,
        in_specs=[
            pl.BlockSpec((BLOCK_N, MODEL_DIM), lambda i: (i, 0)),
            pl.BlockSpec((MODEL_DIM, NUM_EXPERTS), lambda i: (0, 0)),
            pl.BlockSpec((1, NUM_EXPERTS), lambda i: (0, 0)),
        ],
        out_specs=pl.BlockSpec((BLOCK_N, NUM_EXPERTS), lambda i: (i, 0)),
        out_shape=jax.ShapeDtypeStruct((N_TOKENS, NUM_EXPERTS), jnp.float32),
    )(x, W, b2)
